# Initial kernel scaffold; baseline (speedup 1.0000x reference)
import jax
import jax.numpy as jnp
from jax.experimental import pallas as pl


def kernel(x, Wr, br, W1, b1, W2, b2):
    raise NotImplementedError("write your pallas kernel here")


# speedup vs baseline: 3.7654x; 3.7654x over previous
import functools

import jax
import jax.numpy as jnp
from jax.experimental import pallas as pl
from jax.experimental.pallas import tpu as pltpu

B = 2048
D = 768
H = 512
E = 8
K = 2


def _moe_block_kernel(x_ref, wr_ref, br_ref, w1_ref, b1_ref, w2_ref, b2_ref,
                      out_ref):
    xb = x_ref[...]
    logits = jax.lax.dot_general(
        xb, wr_ref[...], (((1,), (1,)), ((), ())),
        preferred_element_type=jnp.float32) + br_ref[...]
    m = jnp.max(logits, axis=-1, keepdims=True)
    ex = jnp.exp(logits - m)
    probs = ex / jnp.sum(ex, axis=-1, keepdims=True)

    eids = jax.lax.broadcasted_iota(jnp.int32, logits.shape, 1)
    i1 = jnp.argmax(logits, axis=-1, keepdims=True)
    masked = jnp.where(eids == i1, -jnp.inf, logits)
    i2 = jnp.argmax(masked, axis=-1, keepdims=True)
    sel = (eids == i1) | (eids == i2)
    scale = jnp.where(sel, probs, 0.0)

    acc = jnp.zeros(out_ref.shape, dtype=jnp.float32)
    for e in range(E):
        h = jax.lax.dot_general(
            xb, w1_ref[e], (((1,), (1,)), ((), ())),
            preferred_element_type=jnp.float32) + b1_ref[e][None, :]
        h = jnp.maximum(h, 0.0)
        y = jax.lax.dot_general(
            h, w2_ref[e], (((1,), (1,)), ((), ())),
            preferred_element_type=jnp.float32) + b2_ref[e][None, :]
        acc = acc + scale[:, e][:, None] * y
    out_ref[...] = acc


@functools.partial(jax.jit, static_argnames=())
def kernel(x, Wr, br, W1, b1, W2, b2):
    BT = 256
    grid = (B // BT,)
    br2 = br.reshape(1, E)
    out = pl.pallas_call(
        _moe_block_kernel,
        grid=grid,
        in_specs=[
            pl.BlockSpec((BT, D), lambda i: (i, 0)),
            pl.BlockSpec((E, D), lambda i: (0, 0)),
            pl.BlockSpec((1, E), lambda i: (0, 0)),
            pl.BlockSpec((E, H, D), lambda i: (0, 0, 0)),
            pl.BlockSpec((E, H), lambda i: (0, 0)),
            pl.BlockSpec((E, D, H), lambda i: (0, 0, 0)),
            pl.BlockSpec((E, D), lambda i: (0, 0)),
        ],
        out_specs=pl.BlockSpec((BT, D), lambda i: (i, 0)),
        out_shape=jax.ShapeDtypeStruct((B, D), jnp.float32),
    )(x, Wr, br2, W1, b1, W2, b2)
    return out
